# baseline (speedup 1.0000x reference)
import functools

import jax
import jax.numpy as jnp
import numpy as np
from jax import lax
from jax.experimental import pallas as pl
from jax.experimental.pallas import tpu as pltpu
from jax.experimental.pallas import tpu_sc as plsc

N = 10000
E = 320000
D = 128
DE = 16

NC = 2
NS = 16
LANES = 16
NW = NC * NS
EPW = E // NW
CH = 80
NCHUNK = EPW // CH
RPS = 624
TAIL_OFF = NS * RPS
TAIL = N - TAIL_OFF

SQRT1_2 = np.float32(0.7071067811865476)
ERF_P = np.float32(0.47047)
ERF_A1 = np.float32(0.3480242)
ERF_A2 = np.float32(-0.0958798)
ERF_A3 = np.float32(0.7478556)
HALF = np.float32(0.5)
ONE = np.float32(1.0)


def _gelu(v):
    y = v * SQRT1_2
    a = jnp.abs(y)
    t = ONE / (ONE + ERF_P * a)
    p = t * (ERF_A1 + t * (ERF_A2 + t * ERF_A3))
    e = jnp.exp(-(y * y))
    pe = p * e
    erf = jnp.where(y < 0.0, pe - ONE, ONE - pe)
    return HALF * v * (ONE + erf)


def _eemb_body(ea_ref, we_ref, be_ref, out_ref):
    out_ref[...] = (
        jnp.dot(ea_ref[...], we_ref[...], preferred_element_type=jnp.float32)
        + be_ref[...]
    )


_EEMB_BLK = 8000


def _eemb(edge_attr, we, be):
    grid = (E // _EEMB_BLK,)
    return pl.pallas_call(
        _eemb_body,
        grid=grid,
        in_specs=[
            pl.BlockSpec((_EEMB_BLK, DE), lambda i: (i, 0)),
            pl.BlockSpec((DE, D), lambda i: (0, 0)),
            pl.BlockSpec((1, D), lambda i: (0, 0)),
        ],
        out_specs=pl.BlockSpec((_EEMB_BLK, D), lambda i: (i, 0)),
        out_shape=jax.ShapeDtypeStruct((E, D), jnp.float32),
    )(edge_attr, we, be.reshape(1, D))


def _edge_body(h_hbm, eemb_hbm, src_hbm, dst_hbm, ew_hbm, zeros_hbm, out_hbm,
               src_v, dst_v, ew_v, rows_v, eemb_v, acc_sh, sem_g, sem_e):
    c = lax.axis_index("c")
    s = lax.axis_index("s")
    wid = s * NC + c
    pltpu.sync_copy(zeros_hbm.at[pl.ds(s * RPS, RPS)],
                    acc_sh.at[pl.ds(s * RPS, RPS)])

    @pl.when(s == NS - 1)
    def _():
        pltpu.sync_copy(zeros_hbm.at[pl.ds(TAIL_OFF, TAIL)],
                        acc_sh.at[pl.ds(TAIL_OFF, TAIL)])

    plsc.subcore_barrier()
    base0 = wid * EPW

    def chunk_body(i, carry):
        base = base0 + i * CH
        pltpu.sync_copy(src_hbm.at[pl.ds(base, CH)], src_v)
        pltpu.sync_copy(dst_hbm.at[pl.ds(base, CH)], dst_v)
        pltpu.sync_copy(ew_hbm.at[pl.ds(base, CH)], ew_v)
        cp_g = pltpu.async_copy(h_hbm.at[src_v], rows_v, sem_g)
        cp_e = pltpu.async_copy(eemb_hbm.at[pl.ds(base, CH)], eemb_v, sem_e)
        cp_g.wait()
        cp_e.wait()

        for g in range(CH // LANES):
            ewg = ew_v[pl.ds(g * LANES, LANES)]

            def row_body(r2, rcarry, ewg=ewg, g=g):
                r = g * LANES + r2
                ws = lax.gather(
                    ewg, jnp.full((LANES, 1), r2, jnp.int32),
                    dimension_numbers=lax.GatherDimensionNumbers(
                        offset_dims=(), collapsed_slice_dims=(0,),
                        start_index_map=(0,)),
                    slice_sizes=(1,),
                    mode=lax.GatherScatterMode.PROMISE_IN_BOUNDS)
                for j in range(D // LANES):
                    sl = pl.ds(j * LANES, LANES)
                    v = rows_v[r, sl] + eemb_v[r, sl]
                    rows_v[r, sl] = _gelu(v) * ws
                return rcarry

            lax.fori_loop(0, LANES, row_body, 0)
        pltpu.sync_copy(rows_v, acc_sh.at[dst_v], add=True)
        return carry

    lax.fori_loop(0, NCHUNK, chunk_body, 0)
    plsc.subcore_barrier()
    pltpu.sync_copy(acc_sh.at[pl.ds(s * RPS, RPS)],
                    out_hbm.at[c, pl.ds(s * RPS, RPS)])

    @pl.when(s == NS - 1)
    def _():
        pltpu.sync_copy(acc_sh.at[pl.ds(TAIL_OFF, TAIL)],
                        out_hbm.at[c, pl.ds(TAIL_OFF, TAIL)])


@functools.cache
def _edge_call():
    return pl.kernel(
        _edge_body,
        out_type=jax.ShapeDtypeStruct((NC, N, D), jnp.float32),
        mesh=plsc.VectorSubcoreMesh(core_axis_name="c", subcore_axis_name="s",
                                    num_cores=NC, num_subcores=NS),
        scratch_types=[
            pltpu.VMEM((CH,), jnp.int32),
            pltpu.VMEM((CH,), jnp.int32),
            pltpu.VMEM((CH,), jnp.float32),
            pltpu.VMEM((CH, D), jnp.float32),
            pltpu.VMEM((CH, D), jnp.float32),
            pltpu.VMEM_SHARED((N, D), jnp.float32),
            pltpu.SemaphoreType.DMA,
            pltpu.SemaphoreType.DMA,
        ],
    )


def _dense_body(h_ref, part_ref, w1_ref, b1_ref, w2_ref, b2_ref,
                gamma_ref, beta_ref, eps_ref, out_ref):
    h = h_ref[...]
    agg = part_ref[0] + part_ref[1]
    z = (ONE + eps_ref[0, 0]) * h + agg
    u = _gelu(jnp.dot(z, w1_ref[...], preferred_element_type=jnp.float32)
              + b1_ref[...])
    z2 = jnp.dot(u, w2_ref[...], preferred_element_type=jnp.float32) + b2_ref[...]
    mean = jnp.mean(z2, axis=0, keepdims=True)
    ctr = z2 - mean
    var = jnp.mean(ctr * ctr, axis=0, keepdims=True)
    zn = ctr * lax.rsqrt(var + np.float32(1e-5)) * gamma_ref[...] + beta_ref[...]
    out_ref[...] = (h + _gelu(zn)) * SQRT1_2


def _dense(h, part, p):
    return pl.pallas_call(
        _dense_body,
        out_shape=jax.ShapeDtypeStruct((N, D), jnp.float32),
    )(h, part, p['W1'], p['b1'].reshape(1, D), p['W2'], p['b2'].reshape(1, D),
      p['gamma'].reshape(1, D), p['beta'].reshape(1, D),
      p['eps'].reshape(1, 1))


def kernel(x, edge_index, edge_attr, edge_weight, params):
    src = edge_index[0]
    dst = edge_index[1]
    e_emb = _eemb(edge_attr, params['We'], params['be'])
    zeros = jnp.zeros((N, D), jnp.float32)
    h = x
    for p in params['layers']:
        part = _edge_call()(h, e_emb, src, dst, edge_weight, zeros)
        h = _dense(h, part, p)
    return h


# speedup vs baseline: 1.2127x; 1.2127x over previous
import functools

import jax
import jax.numpy as jnp
import numpy as np
from jax import lax
from jax.experimental import pallas as pl
from jax.experimental.pallas import tpu as pltpu
from jax.experimental.pallas import tpu_sc as plsc

N = 10000
E = 320000
D = 128
DE = 16

NC = 2
NS = 16
LANES = 16
NW = NC * NS
EPW = E // NW
CH = 40
K = 2
TOTCH = E // CH
CPW = EPW // CH
NGRP = CPW // K
NPAIR = NGRP // 2
RPS = 624
TAIL_OFF = NS * RPS
TAIL = N - TAIL_OFF

SQRT1_2 = np.float32(0.7071067811865476)
ERF_P2 = np.float32(0.47047 * 0.7071067811865476)
ERF_A1 = np.float32(0.3480242)
ERF_A2 = np.float32(-0.0958798)
ERF_A3 = np.float32(0.7478556)
HALF = np.float32(0.5)
ONE = np.float32(1.0)
TWO = np.float32(2.0)
NHALF = np.float32(-0.5)


def _gelu(v):
    av = jnp.abs(v)
    d = ONE + ERF_P2 * av
    t = ONE / d
    t = t * (TWO - d * t)
    p = t * (ERF_A1 + t * (ERF_A2 + t * ERF_A3))
    e = jnp.exp(NHALF * (v * v))
    return HALF * (v + av - av * (p * e))


def _gelu_w(v, hw):
    av = jnp.abs(v)
    d = ONE + ERF_P2 * av
    t = ONE / d
    t = t * (TWO - d * t)
    p = t * (ERF_A1 + t * (ERF_A2 + t * ERF_A3))
    e = jnp.exp(NHALF * (v * v))
    return hw * (v + av - av * (p * e))


def _eemb_body(ea_ref, we_ref, be_ref, out_ref):
    out_ref[...] = (
        jnp.dot(ea_ref[...], we_ref[...], preferred_element_type=jnp.float32)
        + be_ref[...]
    )


_EEMB_BLK = 8000


def _eemb(edge_attr, we, be):
    grid = (E // _EEMB_BLK,)
    return pl.pallas_call(
        _eemb_body,
        grid=grid,
        in_specs=[
            pl.BlockSpec((_EEMB_BLK, DE), lambda i: (i, 0)),
            pl.BlockSpec((DE, D), lambda i: (0, 0)),
            pl.BlockSpec((1, D), lambda i: (0, 0)),
        ],
        out_specs=pl.BlockSpec((_EEMB_BLK, D), lambda i: (i, 0)),
        out_shape=jax.ShapeDtypeStruct((E, D), jnp.float32),
    )(edge_attr, we, be.reshape(1, D))


PKR = 4 * K
TOTG = TOTCH // K


def _make_edge_body():
    def body(h_hbm, eemb_hbm, pk_hbm, zeros_hbm, out_hbm,
             pk_v, rows_v, ee_v, acc_sh,
             sem_g0, sem_g1, sem_e0, sem_e1, sem_s0, sem_s1):
        c = lax.axis_index("c")
        s = lax.axis_index("s")
        wid = s * NC + c
        sem_g = (sem_g0, sem_g1)
        sem_e = (sem_e0, sem_e1)
        sem_s = (sem_s0, sem_s1)
        cb0 = wid * CPW

        pltpu.sync_copy(zeros_hbm.at[pl.ds(s * RPS, RPS)],
                        acc_sh.at[pl.ds(s * RPS, RPS)])

        @pl.when(s == NS - 1)
        def _():
            pltpu.sync_copy(zeros_hbm.at[pl.ds(TAIL_OFF, TAIL)],
                            acc_sh.at[pl.ds(TAIL_OFF, TAIL)])

        plsc.subcore_barrier()

        g0w = wid * NGRP

        def load_pk(g, st):
            pltpu.sync_copy(pk_hbm.at[g0w + g], pk_v.at[st])

        def issue_group(g, st):
            cb = cb0 + g * K
            for k in range(K):
                j = st * K + k
                pltpu.async_copy(h_hbm.at[pk_v.at[st, 4 * k]], rows_v.at[j],
                                 sem_g[st])
                pltpu.async_copy(eemb_hbm.at[pl.ds((cb + k) * CH, CH)],
                                 ee_v.at[j], sem_e[st])

        def wait_group(g, st):
            cb = cb0 + g * K
            for k in range(K):
                j = st * K + k
                pltpu.make_async_copy(h_hbm.at[pk_v.at[st, 4 * k]],
                                      rows_v.at[j], sem_g[st]).wait()
                pltpu.make_async_copy(eemb_hbm.at[pl.ds((cb + k) * CH, CH)],
                                      ee_v.at[j], sem_e[st]).wait()

        def drain_scatters(st):
            for k in range(K):
                j = st * K + k
                pltpu.make_async_copy(rows_v.at[j],
                                      acc_sh.at[pk_v.at[st, 4 * k + 1]],
                                      sem_s[st]).wait()

        def compute_group(st):
            for k in range(K):
                j = st * K + k
                rref = rows_v.at[j]
                eref = ee_v.at[j]

                def do_rows(lo, cnt, ew_off, lane_off, rref=rref, eref=eref,
                            st=st, k=k):
                    ewg = lax.bitcast_convert_type(
                        pk_v[st, 4 * k + 2, pl.ds(ew_off, LANES)],
                        jnp.float32)

                    def row_fn(r2, carry):
                        r = lo + r2
                        ws = lax.gather(
                            ewg, jnp.full((LANES, 1), r2 + lane_off,
                                          jnp.int32),
                            dimension_numbers=lax.GatherDimensionNumbers(
                                offset_dims=(), collapsed_slice_dims=(0,),
                                start_index_map=(0,)),
                            slice_sizes=(1,),
                            mode=lax.GatherScatterMode.PROMISE_IN_BOUNDS)
                        hw = ws * HALF
                        for jj in range(D // LANES):
                            sl = pl.ds(jj * LANES, LANES)
                            v = rref[r, sl] + eref[r, sl]
                            rref[r, sl] = _gelu_w(v, hw)
                        return carry

                    lax.fori_loop(0, cnt, row_fn, 0)

                do_rows(0, 16, 0, 0)
                do_rows(16, 16, 16, 0)
                do_rows(32, 8, 24, 8)
                pltpu.async_copy(rows_v.at[j],
                                 acc_sh.at[pk_v.at[st, 4 * k + 1]],
                                 sem_s[st], add=True)

        load_pk(0, 0)
        issue_group(0, 0)

        def pair_body(t, carry):
            g0 = 2 * t
            @pl.when(t >= 1)
            def _():
                drain_scatters(1)
            load_pk(g0 + 1, 1)
            issue_group(g0 + 1, 1)
            wait_group(g0, 0)
            compute_group(0)
            drain_scatters(0)
            load_pk(g0 + 2, 0)
            issue_group(g0 + 2, 0)
            wait_group(g0 + 1, 1)
            compute_group(1)
            return carry

        lax.fori_loop(0, NPAIR, pair_body, 0)
        wait_group(NGRP - 1, 0)
        compute_group(0)
        drain_scatters(1)
        drain_scatters(0)

        plsc.subcore_barrier()
        pltpu.sync_copy(acc_sh.at[pl.ds(s * RPS, RPS)],
                        out_hbm.at[c, pl.ds(s * RPS, RPS)])

        @pl.when(s == NS - 1)
        def _():
            pltpu.sync_copy(acc_sh.at[pl.ds(TAIL_OFF, TAIL)],
                            out_hbm.at[c, pl.ds(TAIL_OFF, TAIL)])

    return body


@functools.cache
def _edge_call():
    return pl.kernel(
        _make_edge_body(),
        out_type=jax.ShapeDtypeStruct((NC, N, D), jnp.float32),
        mesh=plsc.VectorSubcoreMesh(core_axis_name="c", subcore_axis_name="s",
                                    num_cores=NC, num_subcores=NS),
        scratch_types=[
            pltpu.VMEM((2, PKR, CH), jnp.int32),
            pltpu.VMEM((2 * K, CH, D), jnp.float32),
            pltpu.VMEM((2 * K, CH, D), jnp.float32),
            pltpu.VMEM_SHARED((N, D), jnp.float32),
            pltpu.SemaphoreType.DMA,
            pltpu.SemaphoreType.DMA,
            pltpu.SemaphoreType.DMA,
            pltpu.SemaphoreType.DMA,
            pltpu.SemaphoreType.DMA,
            pltpu.SemaphoreType.DMA,
        ],
    )


def _dense_body(h_ref, part_ref, w1_ref, b1_ref, w2_ref, b2_ref,
                gamma_ref, beta_ref, eps_ref, out_ref):
    h = h_ref[...]
    agg = part_ref[0] + part_ref[1]
    z = (ONE + eps_ref[0, 0]) * h + agg
    u = _gelu(jnp.dot(z, w1_ref[...], preferred_element_type=jnp.float32)
              + b1_ref[...])
    z2 = jnp.dot(u, w2_ref[...], preferred_element_type=jnp.float32) + b2_ref[...]
    mean = jnp.mean(z2, axis=0, keepdims=True)
    ctr = z2 - mean
    var = jnp.mean(ctr * ctr, axis=0, keepdims=True)
    zn = ctr * lax.rsqrt(var + np.float32(1e-5)) * gamma_ref[...] + beta_ref[...]
    out_ref[...] = (h + _gelu(zn)) * SQRT1_2


def _dense(h, part, p):
    return pl.pallas_call(
        _dense_body,
        out_shape=jax.ShapeDtypeStruct((N, D), jnp.float32),
    )(h, part, p['W1'], p['b1'].reshape(1, D), p['W2'], p['b2'].reshape(1, D),
      p['gamma'].reshape(1, D), p['beta'].reshape(1, D),
      p['eps'].reshape(1, 1))


def kernel(x, edge_index, edge_attr, edge_weight, params):
    src = edge_index[0]
    dst = edge_index[1]
    pk = jnp.stack([
        src.reshape(TOTCH, CH),
        dst.reshape(TOTCH, CH),
        lax.bitcast_convert_type(edge_weight, jnp.int32).reshape(TOTCH, CH),
        jnp.zeros((TOTCH, CH), jnp.int32),
    ], axis=1).reshape(TOTG, PKR, CH)
    e_emb = _eemb(edge_attr, params['We'], params['be'])
    zeros = jnp.zeros((N, D), jnp.float32)
    h = x
    for p in params['layers']:
        part = _edge_call()(h, e_emb, pk, zeros)
        h = _dense(h, part, p)
    return h
